# baseline (speedup 1.0000x reference)
import dataclasses

import jax
import jax.numpy as jnp
from jax import lax
from jax.experimental import pallas as pl
from jax.experimental.pallas import tpu as pltpu
from jax.experimental.pallas import tpu_sc as plsc

N = 10000
E = 320000
D = 128
NPAD = 10240
NTILE = 16
CHUNK = E // NTILE
W1 = 80
GROUPS = W1 // 16
BLK = 4000
NBLK = CHUNK // BLK
WPB = BLK // W1
RPT = NPAD // NTILE
BN = 2000
NG = 3
f32 = jnp.float32


def _prep_body(x_ref, w_ref, a1_ref, a2_ref, relp_ref, wr_ref, a3_ref,
               wx_ref, ti_ref, tj_ref, tr_ref):
    xb = x_ref[...]
    wx = jnp.dot(xb, w_ref[...], preferred_element_type=f32)
    wx_ref[...] = wx
    ti_ref[...] = jnp.dot(wx, a1_ref[...], preferred_element_type=f32)
    tj_ref[...] = jnp.dot(wx, a2_ref[...], preferred_element_type=f32)
    relwr = jnp.dot(relp_ref[...], wr_ref[...], preferred_element_type=f32)
    tr_ref[...] = jnp.dot(relwr, a3_ref[...], preferred_element_type=f32)


def _prep(x, p):
    W, Wr, a, _, rel = p
    a1, a2, a3 = a[:D], a[D:2 * D], a[2 * D:]
    relp = jnp.zeros((16, rel.shape[1]), f32).at[:rel.shape[0]].set(rel)
    wx, ti, tj, trv = pl.pallas_call(
        _prep_body,
        grid=(N // BN,),
        in_specs=[
            pl.BlockSpec((BN, D), lambda i: (i, 0)),
            pl.BlockSpec((D, D), lambda i: (0, 0)),
            pl.BlockSpec((D, 1), lambda i: (0, 0)),
            pl.BlockSpec((D, 1), lambda i: (0, 0)),
            pl.BlockSpec((16, relp.shape[1]), lambda i: (0, 0)),
            pl.BlockSpec((relp.shape[1], D), lambda i: (0, 0)),
            pl.BlockSpec((D, 1), lambda i: (0, 0)),
        ],
        out_specs=[
            pl.BlockSpec((BN, D), lambda i: (i, 0)),
            pl.BlockSpec((BN, 1), lambda i: (i, 0)),
            pl.BlockSpec((BN, 1), lambda i: (i, 0)),
            pl.BlockSpec((16, 1), lambda i: (0, 0)),
        ],
        out_shape=[
            jax.ShapeDtypeStruct((N, D), f32),
            jax.ShapeDtypeStruct((N, 1), f32),
            jax.ShapeDtypeStruct((N, 1), f32),
            jax.ShapeDtypeStruct((16, 1), f32),
        ],
    )(x, W, a1, a2, relp, Wr, a3)
    return wx, ti[:, 0], tj[:, 0], trv[:, 0]


def _post_body(agg_ref, x_ref, wres_ref, o_ref):
    v = agg_ref[...] + jnp.dot(x_ref[...], wres_ref[...],
                               preferred_element_type=f32)
    ev = jnp.exp(jnp.minimum(v, 0.0)) - 1.0
    v = jnp.where(v > 0, v, ev)
    o_ref[...] = jnp.maximum(v, 0.01 * v)


def _post(agg, x, wres):
    return pl.pallas_call(
        _post_body,
        grid=(N // BN,),
        in_specs=[
            pl.BlockSpec((BN, D), lambda i: (i, 0)),
            pl.BlockSpec((BN, D), lambda i: (i, 0)),
            pl.BlockSpec((D, D), lambda i: (0, 0)),
        ],
        out_specs=pl.BlockSpec((BN, D), lambda i: (i, 0)),
        out_shape=jax.ShapeDtypeStruct((N, D), f32),
    )(agg, x, wres)


def _inv_body(s_ref, o_ref):
    o_ref[...] = 1.0 / (s_ref[...] + 1e-16)


def _inv(s2):
    return pl.pallas_call(
        _inv_body,
        out_shape=jax.ShapeDtypeStruct((NG * 2 * NPAD,), f32),
    )(s2)


def _sc_params():
    cp = pltpu.CompilerParams()
    if "needs_layout_passes" in pltpu.CompilerParams.__dataclass_fields__:
        cp = dataclasses.replace(cp, needs_layout_passes=False)
    return cp


def _sca_body(src_hbm, dst_hbm, et_hbm, ti_hbm, tj_hbm, tr_hbm,
              s_out, e_out,
              ti_v, tj_v, tr_v, srcb_v, dstb_v, etb_v, eb_v, zero_v,
              s_sh, isem, asem):
    c = lax.axis_index("c")
    sid = lax.axis_index("s")
    zvec = jnp.zeros((16,), f32)

    @pl.loop(0, 5)
    def _(i):
        zero_v[pl.ds(i * 16, 16)] = zvec

    for g in range(NG):
        @pl.loop(0, RPT // W1)
        def _(j):
            pltpu.sync_copy(
                zero_v, s_sh.at[pl.ds(g * NPAD + sid * RPT + j * W1, W1)])

    plsc.subcore_barrier()

    for g in range(NG):
        pltpu.sync_copy(ti_hbm.at[g, c], ti_v)
        pltpu.sync_copy(tj_hbm.at[g, c], tj_v)
        pltpu.sync_copy(tr_hbm.at[g, c], tr_v)
        gP = g * NPAD

        @pl.loop(0, NBLK)
        def _(b):
            c1 = pltpu.async_copy(src_hbm.at[g, sid, b], srcb_v, isem)
            c2 = pltpu.async_copy(dst_hbm.at[g, sid, b], dstb_v, isem)
            c3 = pltpu.async_copy(et_hbm.at[g, sid, b], etb_v, isem)
            c1.wait()
            c2.wait()
            c3.wait()

            @pl.loop(0, WPB)
            def _(w):
                for k in range(GROUPS):
                    sl = pl.ds(k * 16, 16)
                    d16 = dstb_v[w, sl]
                    al = (plsc.load_gather(ti_v, [d16])
                          + plsc.load_gather(tj_v, [srcb_v[w, sl]])
                          + plsc.load_gather(tr_v, [etb_v[w, sl]]))
                    al = jnp.maximum(al, 0.2 * al)
                    eb_v[w, sl] = jnp.exp(al)
                    dstb_v[w, sl] = d16 + gP
                pltpu.async_copy(eb_v.at[w], s_sh.at[dstb_v.at[w]], asem,
                                 add=True)

            pltpu.sync_copy(eb_v, e_out.at[g, c, sid, b])

            @pl.loop(0, WPB)
            def _(w):
                pltpu.make_async_copy(eb_v.at[w], s_sh.at[dstb_v.at[w]],
                                     asem).wait()

    plsc.subcore_barrier()

    for g in range(NG):
        pltpu.sync_copy(
            s_sh.at[pl.ds(g * NPAD + sid * RPT, RPT)],
            s_out.at[pl.ds((2 * g + c) * NPAD + sid * RPT, RPT)])


def _sc_a(src5, dst5, et5, ti3, tj3, tr3):
    mesh = plsc.VectorSubcoreMesh(core_axis_name="c", subcore_axis_name="s")
    kern = pl.kernel(
        _sca_body,
        compiler_params=_sc_params(),
        out_type=(
            jax.ShapeDtypeStruct((NG * 2 * NPAD,), f32),
            jax.ShapeDtypeStruct((NG, 2, NTILE, NBLK, WPB, W1), f32),
        ),
        mesh=mesh,
        scratch_types=[
            pltpu.VMEM((N,), f32),
            pltpu.VMEM((N,), f32),
            pltpu.VMEM((16,), f32),
            pltpu.VMEM((WPB, W1), jnp.int32),
            pltpu.VMEM((WPB, W1), jnp.int32),
            pltpu.VMEM((WPB, W1), jnp.int32),
            pltpu.VMEM((WPB, W1), f32),
            pltpu.VMEM((W1,), f32),
            pltpu.VMEM_SHARED((NG * NPAD,), f32),
            pltpu.SemaphoreType.DMA,
            pltpu.SemaphoreType.DMA,
        ],
    )
    return kern(src5, dst5, et5, ti3, tj3, tr3)


def _scb_body(src_hbm, dst_hbm, inv_hbm, e_hbm, wx_hbm,
              out_hbm,
              srcb_v, dstb_v, eb_v, dstw_a, dstw_b, invw_a, invw_b,
              rows_a, rows_b,
              agg_sh, isem, gsem_a, gsem_b, ssem_a, ssem_b):
    c = lax.axis_index("c")
    sid = lax.axis_index("s")
    zvec = jnp.zeros((16,), f32)

    def _scale(w, invw, rows):
        for k in range(GROUPS):
            sl = pl.ds(k * 16, 16)
            eb_v[w, sl] = eb_v[w, sl] * invw[sl]
        w16 = jnp.broadcast_to(w, (16,))

        @pl.loop(0, W1, unroll=4)
        def _(r):
            sv16 = plsc.load_gather(eb_v, [w16, jnp.broadcast_to(r, (16,))])
            for q in range(D // 16):
                rsl = pl.ds(q * 16, 16)
                rows[r, rsl] = rows[r, rsl] * sv16

    for g in range(NG):
        lN = (2 * g + c) * N
        lP = (2 * g + c) * NPAD

        @pl.loop(0, W1)
        def _(r):
            for q in range(D // 16):
                rows_a[r, pl.ds(q * 16, 16)] = zvec

        @pl.when(sid < 10)
        def _():
            @pl.loop(0, 12)
            def _(j):
                pltpu.sync_copy(rows_a,
                                agg_sh.at[pl.ds(sid * 1000 + j * W1, W1)])

            pltpu.sync_copy(rows_a.at[pl.ds(0, 40)],
                            agg_sh.at[pl.ds(sid * 1000 + 960, 40)])

        plsc.subcore_barrier()

        def _fire(w, dstw, invw, rows, gsem):
            for k in range(GROUPS):
                sl = pl.ds(k * 16, 16)
                dstw[sl] = dstb_v[w, sl] + lP
            pltpu.async_copy(inv_hbm.at[dstw], invw, gsem)
            pltpu.async_copy(wx_hbm.at[srcb_v.at[w]], rows, gsem)

        def _wait(w, dstw, invw, rows, gsem):
            pltpu.make_async_copy(inv_hbm.at[dstw], invw, gsem).wait()
            pltpu.make_async_copy(wx_hbm.at[srcb_v.at[w]], rows, gsem).wait()

        @pl.loop(0, NBLK)
        def _(b):
            c1 = pltpu.async_copy(src_hbm.at[g, sid, b], srcb_v, isem)
            c2 = pltpu.async_copy(dst_hbm.at[g, sid, b], dstb_v, isem)
            c3 = pltpu.async_copy(e_hbm.at[g, c, sid, b], eb_v, isem)
            c1.wait()
            c2.wait()
            c3.wait()

            @pl.loop(0, WPB)
            def _(w):
                for k in range(GROUPS):
                    sl = pl.ds(k * 16, 16)
                    srcb_v[w, sl] = srcb_v[w, sl] + lN

            _fire(0, dstw_a, invw_a, rows_a, gsem_a)

            @pl.loop(0, WPB // 2)
            def _(i):
                w0 = 2 * i
                _fire(w0 + 1, dstw_b, invw_b, rows_b, gsem_b)
                _wait(w0, dstw_a, invw_a, rows_a, gsem_a)
                _scale(w0, invw_a, rows_a)
                pltpu.async_copy(rows_a, agg_sh.at[dstb_v.at[w0]], ssem_a,
                                 add=True)
                _wait(w0 + 1, dstw_b, invw_b, rows_b, gsem_b)
                _scale(w0 + 1, invw_b, rows_b)
                pltpu.async_copy(rows_b, agg_sh.at[dstb_v.at[w0 + 1]], ssem_b,
                                 add=True)
                pltpu.make_async_copy(rows_a, agg_sh.at[dstb_v.at[w0]],
                                      ssem_a).wait()

                @pl.when(i + 1 < WPB // 2)
                def _():
                    _fire(w0 + 2, dstw_a, invw_a, rows_a, gsem_a)

                pltpu.make_async_copy(rows_b, agg_sh.at[dstb_v.at[w0 + 1]],
                                      ssem_b).wait()

        plsc.subcore_barrier()

        @pl.when(sid < 10)
        def _():
            @pl.loop(0, 5)
            def _(j):
                off = sid * 1000 + j * 200
                pltpu.sync_copy(agg_sh.at[pl.ds(off, 200)],
                                out_hbm.at[pl.ds(lN + off, 200)])

        plsc.subcore_barrier()


def _sc_b(src5, dst5, invf, e6, wxf):
    mesh = plsc.VectorSubcoreMesh(core_axis_name="c", subcore_axis_name="s")
    kern = pl.kernel(
        _scb_body,
        compiler_params=_sc_params(),
        out_type=jax.ShapeDtypeStruct((NG * 2 * N, D), f32),
        mesh=mesh,
        scratch_types=[
            pltpu.VMEM((WPB, W1), jnp.int32),
            pltpu.VMEM((WPB, W1), jnp.int32),
            pltpu.VMEM((WPB, W1), f32),
            pltpu.VMEM((W1,), jnp.int32),
            pltpu.VMEM((W1,), jnp.int32),
            pltpu.VMEM((W1,), f32),
            pltpu.VMEM((W1,), f32),
            pltpu.VMEM((W1, D), f32),
            pltpu.VMEM((W1, D), f32),
            pltpu.VMEM_SHARED((N, D), f32),
            pltpu.SemaphoreType.DMA,
            pltpu.SemaphoreType.DMA,
            pltpu.SemaphoreType.DMA,
            pltpu.SemaphoreType.DMA,
            pltpu.SemaphoreType.DMA,
        ],
    )
    return kern(src5, dst5, invf, e6, wxf)


def kernel(sub1_text, sub1_struct, sub2_text, sub2_meta, sub3_text, sub3_meta,
           edge_index1, edge_index2, edge_index3,
           edge_type1, edge_type2, edge_type3, params):
    xs = [sub1_text, sub1_struct, sub2_text, sub2_meta, sub3_text, sub3_meta]
    ps = [params[0], params[3], params[1], params[4], params[2], params[5]]
    eis = [edge_index1, edge_index2, edge_index3]
    ets = [edge_type1, edge_type2, edge_type3]

    preps = [_prep(x, p) for x, p in zip(xs, ps)]

    src5 = jnp.stack([ei[0].reshape(NTILE, NBLK, WPB, W1) for ei in eis])
    dst5 = jnp.stack([ei[1].reshape(NTILE, NBLK, WPB, W1) for ei in eis])
    et5 = jnp.stack([et.reshape(NTILE, NBLK, WPB, W1) for et in ets])
    ti3 = jnp.stack([jnp.stack([preps[2 * g][1], preps[2 * g + 1][1]])
                     for g in range(NG)])
    tj3 = jnp.stack([jnp.stack([preps[2 * g][2], preps[2 * g + 1][2]])
                     for g in range(NG)])
    tr3 = jnp.stack([jnp.stack([preps[2 * g][3], preps[2 * g + 1][3]])
                     for g in range(NG)])
    wxf = jnp.concatenate([preps[i][0] for i in range(6)], axis=0)

    sflat, e6 = _sc_a(src5, dst5, et5, ti3, tj3, tr3)
    invf = _inv(sflat)
    aggflat = _sc_b(src5, dst5, invf, e6, wxf)

    outs = tuple(_post(aggflat[i * N:(i + 1) * N], xs[i], ps[i][3])
                 for i in range(6))
    return outs


# speedup vs baseline: 1.0860x; 1.0860x over previous
import dataclasses

import jax
import jax.numpy as jnp
from jax import lax
from jax.experimental import pallas as pl
from jax.experimental.pallas import tpu as pltpu
from jax.experimental.pallas import tpu_sc as plsc

N = 10000
E = 320000
D = 128
NPAD = 10240
NTILE = 16
CHUNK = E // NTILE
W1 = 80
GROUPS = W1 // 16
BLK = 4000
NBLK = CHUNK // BLK
WPB = BLK // W1
RPT = NPAD // NTILE
BN = 2000
NG = 3
f32 = jnp.float32


def _prep_body(x_ref, w_ref, a1_ref, a2_ref, relp_ref, wr_ref, a3_ref,
               wx_ref, ti_ref, tj_ref, tr_ref):
    xb = x_ref[...]
    wx = jnp.dot(xb, w_ref[...], preferred_element_type=f32)
    wx_ref[...] = wx
    ti_ref[...] = jnp.dot(wx, a1_ref[...], preferred_element_type=f32)
    tj_ref[...] = jnp.dot(wx, a2_ref[...], preferred_element_type=f32)
    relwr = jnp.dot(relp_ref[...], wr_ref[...], preferred_element_type=f32)
    tr_ref[...] = jnp.dot(relwr, a3_ref[...], preferred_element_type=f32)


def _prep(x, p):
    W, Wr, a, _, rel = p
    a1, a2, a3 = a[:D], a[D:2 * D], a[2 * D:]
    relp = jnp.zeros((16, rel.shape[1]), f32).at[:rel.shape[0]].set(rel)
    wx, ti, tj, trv = pl.pallas_call(
        _prep_body,
        grid=(N // BN,),
        in_specs=[
            pl.BlockSpec((BN, D), lambda i: (i, 0)),
            pl.BlockSpec((D, D), lambda i: (0, 0)),
            pl.BlockSpec((D, 1), lambda i: (0, 0)),
            pl.BlockSpec((D, 1), lambda i: (0, 0)),
            pl.BlockSpec((16, relp.shape[1]), lambda i: (0, 0)),
            pl.BlockSpec((relp.shape[1], D), lambda i: (0, 0)),
            pl.BlockSpec((D, 1), lambda i: (0, 0)),
        ],
        out_specs=[
            pl.BlockSpec((BN, D), lambda i: (i, 0)),
            pl.BlockSpec((BN, 1), lambda i: (i, 0)),
            pl.BlockSpec((BN, 1), lambda i: (i, 0)),
            pl.BlockSpec((16, 1), lambda i: (0, 0)),
        ],
        out_shape=[
            jax.ShapeDtypeStruct((N, D), f32),
            jax.ShapeDtypeStruct((N, 1), f32),
            jax.ShapeDtypeStruct((N, 1), f32),
            jax.ShapeDtypeStruct((16, 1), f32),
        ],
    )(x, W, a1, a2, relp, Wr, a3)
    return wx, ti[:, 0], tj[:, 0], trv[:, 0]


def _post_body(agg_ref, x_ref, wres_ref, o_ref):
    v = agg_ref[...] + jnp.dot(x_ref[...], wres_ref[...],
                               preferred_element_type=f32)
    ev = jnp.exp(jnp.minimum(v, 0.0)) - 1.0
    v = jnp.where(v > 0, v, ev)
    o_ref[...] = jnp.maximum(v, 0.01 * v)


def _post(agg, x, wres):
    return pl.pallas_call(
        _post_body,
        grid=(N // BN,),
        in_specs=[
            pl.BlockSpec((BN, D), lambda i: (i, 0)),
            pl.BlockSpec((BN, D), lambda i: (i, 0)),
            pl.BlockSpec((D, D), lambda i: (0, 0)),
        ],
        out_specs=pl.BlockSpec((BN, D), lambda i: (i, 0)),
        out_shape=jax.ShapeDtypeStruct((N, D), f32),
    )(agg, x, wres)


def _inv_body(s_ref, o_ref):
    o_ref[...] = 1.0 / (s_ref[...] + 1e-16)


def _inv(s2):
    return pl.pallas_call(
        _inv_body,
        out_shape=jax.ShapeDtypeStruct((2 * NPAD,), f32),
    )(s2)


def _sc_params():
    cp = pltpu.CompilerParams()
    if "needs_layout_passes" in pltpu.CompilerParams.__dataclass_fields__:
        cp = dataclasses.replace(cp, needs_layout_passes=False)
    return cp


def _sca_body(src_hbm, dst_hbm, et_hbm, ti_hbm, tj_hbm, tr_hbm,
              s_out, e_out,
              ti_v, tj_v, tr_v, srcb_v, dstb_v, etb_v, eb_v, zero_v,
              s_sh, isem, asem):
    c = lax.axis_index("c")
    sid = lax.axis_index("s")
    zvec = jnp.zeros((16,), f32)

    pltpu.sync_copy(ti_hbm.at[c], ti_v)
    pltpu.sync_copy(tj_hbm.at[c], tj_v)
    pltpu.sync_copy(tr_hbm.at[c], tr_v)

    @pl.loop(0, 5)
    def _(i):
        zero_v[pl.ds(i * 16, 16)] = zvec

    @pl.loop(0, RPT // W1)
    def _(j):
        pltpu.sync_copy(zero_v, s_sh.at[pl.ds(sid * RPT + j * W1, W1)])

    plsc.subcore_barrier()

    @pl.loop(0, NBLK)
    def _(b):
        c1 = pltpu.async_copy(src_hbm.at[sid, b], srcb_v, isem)
        c2 = pltpu.async_copy(dst_hbm.at[sid, b], dstb_v, isem)
        c3 = pltpu.async_copy(et_hbm.at[sid, b], etb_v, isem)
        c1.wait()
        c2.wait()
        c3.wait()

        @pl.loop(0, WPB)
        def _(w):
            for k in range(GROUPS):
                sl = pl.ds(k * 16, 16)
                al = (plsc.load_gather(ti_v, [dstb_v[w, sl]])
                      + plsc.load_gather(tj_v, [srcb_v[w, sl]])
                      + plsc.load_gather(tr_v, [etb_v[w, sl]]))
                al = jnp.maximum(al, 0.2 * al)
                eb_v[w, sl] = jnp.exp(al)
            pltpu.async_copy(eb_v.at[w], s_sh.at[dstb_v.at[w]], asem,
                             add=True)

        pltpu.sync_copy(eb_v, e_out.at[c, sid, b])

        @pl.loop(0, WPB)
        def _(w):
            pltpu.make_async_copy(eb_v.at[w], s_sh.at[dstb_v.at[w]],
                                 asem).wait()

    plsc.subcore_barrier()

    pltpu.sync_copy(s_sh.at[pl.ds(sid * RPT, RPT)],
                    s_out.at[pl.ds(c * NPAD + sid * RPT, RPT)])


def _sc_a(src4, dst4, et4, ti2, tj2, tr2):
    mesh = plsc.VectorSubcoreMesh(core_axis_name="c", subcore_axis_name="s")
    kern = pl.kernel(
        _sca_body,
        compiler_params=_sc_params(),
        out_type=(
            jax.ShapeDtypeStruct((2 * NPAD,), f32),
            jax.ShapeDtypeStruct((2, NTILE, NBLK, WPB, W1), f32),
        ),
        mesh=mesh,
        scratch_types=[
            pltpu.VMEM((N,), f32),
            pltpu.VMEM((N,), f32),
            pltpu.VMEM((16,), f32),
            pltpu.VMEM((WPB, W1), jnp.int32),
            pltpu.VMEM((WPB, W1), jnp.int32),
            pltpu.VMEM((WPB, W1), jnp.int32),
            pltpu.VMEM((WPB, W1), f32),
            pltpu.VMEM((W1,), f32),
            pltpu.VMEM_SHARED((NPAD,), f32),
            pltpu.SemaphoreType.DMA,
            pltpu.SemaphoreType.DMA,
        ],
    )
    return kern(src4, dst4, et4, ti2, tj2, tr2)


def _scb_body(src_hbm, dst_hbm, inv_hbm, e_hbm, wx_hbm,
              out_hbm,
              srcb_v, dstb_v, eb_v, dstw_a, dstw_b, invw_a, invw_b,
              rows_a, rows_b,
              agg_sh, isem, gsem_a, gsem_b, ssem_a, ssem_b):
    c = lax.axis_index("c")
    sid = lax.axis_index("s")
    zvec = jnp.zeros((16,), f32)
    lN = c * N
    lP = c * NPAD

    def _scale(w, invw, rows):
        for k in range(GROUPS):
            sl = pl.ds(k * 16, 16)
            eb_v[w, sl] = eb_v[w, sl] * invw[sl]
        w16 = jnp.broadcast_to(w, (16,))

        @pl.loop(0, W1, unroll=4)
        def _(r):
            sv16 = plsc.load_gather(eb_v, [w16, jnp.broadcast_to(r, (16,))])
            for q in range(D // 16):
                rsl = pl.ds(q * 16, 16)
                rows[r, rsl] = rows[r, rsl] * sv16

    @pl.loop(0, W1)
    def _(r):
        for q in range(D // 16):
            rows_a[r, pl.ds(q * 16, 16)] = zvec

    @pl.when(sid < 10)
    def _():
        @pl.loop(0, 12)
        def _(j):
            pltpu.sync_copy(rows_a, agg_sh.at[pl.ds(sid * 1000 + j * W1, W1)])

        pltpu.sync_copy(rows_a.at[pl.ds(0, 40)],
                        agg_sh.at[pl.ds(sid * 1000 + 960, 40)])

    plsc.subcore_barrier()

    def _fire(w, dstw, invw, rows, gsem):
        for k in range(GROUPS):
            sl = pl.ds(k * 16, 16)
            dstw[sl] = dstb_v[w, sl] + lP
        pltpu.async_copy(inv_hbm.at[dstw], invw, gsem)
        pltpu.async_copy(wx_hbm.at[srcb_v.at[w]], rows, gsem)

    def _wait(w, dstw, invw, rows, gsem):
        pltpu.make_async_copy(inv_hbm.at[dstw], invw, gsem).wait()
        pltpu.make_async_copy(wx_hbm.at[srcb_v.at[w]], rows, gsem).wait()

    @pl.loop(0, NBLK)
    def _(b):
        c1 = pltpu.async_copy(src_hbm.at[sid, b], srcb_v, isem)
        c2 = pltpu.async_copy(dst_hbm.at[sid, b], dstb_v, isem)
        c3 = pltpu.async_copy(e_hbm.at[c, sid, b], eb_v, isem)
        c1.wait()
        c2.wait()
        c3.wait()

        @pl.loop(0, WPB)
        def _(w):
            for k in range(GROUPS):
                sl = pl.ds(k * 16, 16)
                srcb_v[w, sl] = srcb_v[w, sl] + lN

        _fire(0, dstw_a, invw_a, rows_a, gsem_a)

        @pl.loop(0, WPB // 2)
        def _(i):
            w0 = 2 * i
            _fire(w0 + 1, dstw_b, invw_b, rows_b, gsem_b)
            _wait(w0, dstw_a, invw_a, rows_a, gsem_a)
            _scale(w0, invw_a, rows_a)
            pltpu.async_copy(rows_a, agg_sh.at[dstb_v.at[w0]], ssem_a,
                             add=True)
            _wait(w0 + 1, dstw_b, invw_b, rows_b, gsem_b)
            _scale(w0 + 1, invw_b, rows_b)
            pltpu.async_copy(rows_b, agg_sh.at[dstb_v.at[w0 + 1]], ssem_b,
                             add=True)
            pltpu.make_async_copy(rows_a, agg_sh.at[dstb_v.at[w0]],
                                  ssem_a).wait()

            @pl.when(i + 1 < WPB // 2)
            def _():
                _fire(w0 + 2, dstw_a, invw_a, rows_a, gsem_a)

            pltpu.make_async_copy(rows_b, agg_sh.at[dstb_v.at[w0 + 1]],
                                  ssem_b).wait()

    plsc.subcore_barrier()

    @pl.when(sid < 10)
    def _():
        @pl.loop(0, 5)
        def _(j):
            off = sid * 1000 + j * 200
            pltpu.sync_copy(agg_sh.at[pl.ds(off, 200)],
                            out_hbm.at[pl.ds(c * N + off, 200)])


def _sc_b(src4, dst4, inv2, e4, wx2):
    mesh = plsc.VectorSubcoreMesh(core_axis_name="c", subcore_axis_name="s")
    kern = pl.kernel(
        _scb_body,
        compiler_params=_sc_params(),
        out_type=jax.ShapeDtypeStruct((2 * N, D), f32),
        mesh=mesh,
        scratch_types=[
            pltpu.VMEM((WPB, W1), jnp.int32),
            pltpu.VMEM((WPB, W1), jnp.int32),
            pltpu.VMEM((WPB, W1), f32),
            pltpu.VMEM((W1,), jnp.int32),
            pltpu.VMEM((W1,), jnp.int32),
            pltpu.VMEM((W1,), f32),
            pltpu.VMEM((W1,), f32),
            pltpu.VMEM((W1, D), f32),
            pltpu.VMEM((W1, D), f32),
            pltpu.VMEM_SHARED((N, D), f32),
            pltpu.SemaphoreType.DMA,
            pltpu.SemaphoreType.DMA,
            pltpu.SemaphoreType.DMA,
            pltpu.SemaphoreType.DMA,
            pltpu.SemaphoreType.DMA,
        ],
    )
    return kern(src4, dst4, inv2, e4, wx2)


def kernel(sub1_text, sub1_struct, sub2_text, sub2_meta, sub3_text, sub3_meta,
           edge_index1, edge_index2, edge_index3,
           edge_type1, edge_type2, edge_type3, params):
    pairs = [
        (sub1_text, params[0], sub1_struct, params[3], edge_index1, edge_type1),
        (sub2_text, params[1], sub2_meta, params[4], edge_index2, edge_type2),
        (sub3_text, params[2], sub3_meta, params[5], edge_index3, edge_type3),
    ]
    outs = []
    for xa, pa, xb, pb, ei, et in pairs:
        wxa, tia, tja, tra = _prep(xa, pa)
        wxb, tib, tjb, trb = _prep(xb, pb)
        src4 = ei[0].reshape(NTILE, NBLK, WPB, W1)
        dst4 = ei[1].reshape(NTILE, NBLK, WPB, W1)
        et4 = et.reshape(NTILE, NBLK, WPB, W1)
        ti2 = jnp.stack([tia, tib])
        tj2 = jnp.stack([tja, tjb])
        tr2 = jnp.stack([tra, trb])
        wx2 = jnp.concatenate([wxa, wxb], axis=0)
        s2, e4 = _sc_a(src4, dst4, et4, ti2, tj2, tr2)
        inv2 = _inv(s2)
        aggflat = _sc_b(src4, dst4, inv2, e4, wx2)
        oa = _post(aggflat[:N], xa, pa[3])
        ob = _post(aggflat[N:2 * N], xb, pb[3])
        outs.append((oa, ob))
    return (outs[0][0], outs[0][1], outs[1][0], outs[1][1],
            outs[2][0], outs[2][1])


# speedup vs baseline: 1.0864x; 1.0003x over previous
import dataclasses

import jax
import jax.numpy as jnp
from jax import lax
from jax.experimental import pallas as pl
from jax.experimental.pallas import tpu as pltpu
from jax.experimental.pallas import tpu_sc as plsc

N = 10000
E = 320000
D = 128
NPAD = 10240
NTILE = 16
CHUNK = E // NTILE
W1 = 80
GROUPS = W1 // 16
BLK = 4000
NBLK = CHUNK // BLK
WPB = BLK // W1
RPT = NPAD // NTILE
BN = 2000
NG = 3
f32 = jnp.float32


def _prep_body(x_ref, w_ref, a1_ref, a2_ref, relp_ref, wr_ref, a3_ref,
               wx_ref, ti_ref, tj_ref, tr_ref):
    xb = x_ref[...]
    wx = jnp.dot(xb, w_ref[...], preferred_element_type=f32)
    wx_ref[...] = wx
    ti_ref[...] = jnp.dot(wx, a1_ref[...], preferred_element_type=f32)
    tj_ref[...] = jnp.dot(wx, a2_ref[...], preferred_element_type=f32)
    relwr = jnp.dot(relp_ref[...], wr_ref[...], preferred_element_type=f32)
    tr_ref[...] = jnp.dot(relwr, a3_ref[...], preferred_element_type=f32)


def _prep(x, p):
    W, Wr, a, _, rel = p
    a1, a2, a3 = a[:D], a[D:2 * D], a[2 * D:]
    relp = jnp.zeros((16, rel.shape[1]), f32).at[:rel.shape[0]].set(rel)
    wx, ti, tj, trv = pl.pallas_call(
        _prep_body,
        grid=(N // BN,),
        in_specs=[
            pl.BlockSpec((BN, D), lambda i: (i, 0)),
            pl.BlockSpec((D, D), lambda i: (0, 0)),
            pl.BlockSpec((D, 1), lambda i: (0, 0)),
            pl.BlockSpec((D, 1), lambda i: (0, 0)),
            pl.BlockSpec((16, relp.shape[1]), lambda i: (0, 0)),
            pl.BlockSpec((relp.shape[1], D), lambda i: (0, 0)),
            pl.BlockSpec((D, 1), lambda i: (0, 0)),
        ],
        out_specs=[
            pl.BlockSpec((BN, D), lambda i: (i, 0)),
            pl.BlockSpec((BN, 1), lambda i: (i, 0)),
            pl.BlockSpec((BN, 1), lambda i: (i, 0)),
            pl.BlockSpec((16, 1), lambda i: (0, 0)),
        ],
        out_shape=[
            jax.ShapeDtypeStruct((N, D), f32),
            jax.ShapeDtypeStruct((N, 1), f32),
            jax.ShapeDtypeStruct((N, 1), f32),
            jax.ShapeDtypeStruct((16, 1), f32),
        ],
    )(x, W, a1, a2, relp, Wr, a3)
    return wx, ti[:, 0], tj[:, 0], trv[:, 0]


def _post_body(agg_ref, x_ref, wres_ref, o_ref):
    v = agg_ref[...] + jnp.dot(x_ref[...], wres_ref[...],
                               preferred_element_type=f32)
    ev = jnp.exp(jnp.minimum(v, 0.0)) - 1.0
    v = jnp.where(v > 0, v, ev)
    o_ref[...] = jnp.maximum(v, 0.01 * v)


def _post(agg, x, wres):
    return pl.pallas_call(
        _post_body,
        grid=(N // BN,),
        in_specs=[
            pl.BlockSpec((BN, D), lambda i: (i, 0)),
            pl.BlockSpec((BN, D), lambda i: (i, 0)),
            pl.BlockSpec((D, D), lambda i: (0, 0)),
        ],
        out_specs=pl.BlockSpec((BN, D), lambda i: (i, 0)),
        out_shape=jax.ShapeDtypeStruct((N, D), f32),
    )(agg, x, wres)


def _inv_body(s_ref, o_ref):
    o_ref[...] = 1.0 / (s_ref[...] + 1e-16)


def _inv(s2):
    return pl.pallas_call(
        _inv_body,
        out_shape=jax.ShapeDtypeStruct((2 * NPAD,), f32),
    )(s2)


def _sc_params():
    cp = pltpu.CompilerParams()
    if "needs_layout_passes" in pltpu.CompilerParams.__dataclass_fields__:
        cp = dataclasses.replace(cp, needs_layout_passes=False)
    return cp


def _sca_body(src_hbm, dst_hbm, et_hbm, ti_hbm, tj_hbm, tr_hbm,
              s_out, e_out,
              ti_v, tj_v, tr_v, srcb_v, dstb_v, etb_v, eb_v, zero_v,
              s_sh, isem, asem):
    c = lax.axis_index("c")
    sid = lax.axis_index("s")
    zvec = jnp.zeros((16,), f32)

    pltpu.sync_copy(ti_hbm.at[c], ti_v)
    pltpu.sync_copy(tj_hbm.at[c], tj_v)
    pltpu.sync_copy(tr_hbm.at[c], tr_v)

    @pl.loop(0, 5)
    def _(i):
        zero_v[pl.ds(i * 16, 16)] = zvec

    @pl.loop(0, RPT // W1)
    def _(j):
        pltpu.sync_copy(zero_v, s_sh.at[pl.ds(sid * RPT + j * W1, W1)])

    plsc.subcore_barrier()

    @pl.loop(0, NBLK)
    def _(b):
        c1 = pltpu.async_copy(src_hbm.at[sid, b], srcb_v, isem)
        c2 = pltpu.async_copy(dst_hbm.at[sid, b], dstb_v, isem)
        c3 = pltpu.async_copy(et_hbm.at[sid, b], etb_v, isem)
        c1.wait()
        c2.wait()
        c3.wait()

        @pl.loop(0, WPB)
        def _(w):
            for k in range(GROUPS):
                sl = pl.ds(k * 16, 16)
                al = (plsc.load_gather(ti_v, [dstb_v[w, sl]])
                      + plsc.load_gather(tj_v, [srcb_v[w, sl]])
                      + plsc.load_gather(tr_v, [etb_v[w, sl]]))
                al = jnp.maximum(al, 0.2 * al)
                eb_v[w, sl] = jnp.exp(al)
            pltpu.async_copy(eb_v.at[w], s_sh.at[dstb_v.at[w]], asem,
                             add=True)

        pltpu.sync_copy(eb_v, e_out.at[c, sid, b])

        @pl.loop(0, WPB)
        def _(w):
            pltpu.make_async_copy(eb_v.at[w], s_sh.at[dstb_v.at[w]],
                                 asem).wait()

    plsc.subcore_barrier()

    pltpu.sync_copy(s_sh.at[pl.ds(sid * RPT, RPT)],
                    s_out.at[pl.ds(c * NPAD + sid * RPT, RPT)])


def _sc_a(src4, dst4, et4, ti2, tj2, tr2):
    mesh = plsc.VectorSubcoreMesh(core_axis_name="c", subcore_axis_name="s")
    kern = pl.kernel(
        _sca_body,
        compiler_params=_sc_params(),
        out_type=(
            jax.ShapeDtypeStruct((2 * NPAD,), f32),
            jax.ShapeDtypeStruct((2, NTILE, NBLK, WPB, W1), f32),
        ),
        mesh=mesh,
        scratch_types=[
            pltpu.VMEM((N,), f32),
            pltpu.VMEM((N,), f32),
            pltpu.VMEM((16,), f32),
            pltpu.VMEM((WPB, W1), jnp.int32),
            pltpu.VMEM((WPB, W1), jnp.int32),
            pltpu.VMEM((WPB, W1), jnp.int32),
            pltpu.VMEM((WPB, W1), f32),
            pltpu.VMEM((W1,), f32),
            pltpu.VMEM_SHARED((NPAD,), f32),
            pltpu.SemaphoreType.DMA,
            pltpu.SemaphoreType.DMA,
        ],
    )
    return kern(src4, dst4, et4, ti2, tj2, tr2)


def _scb_body(src_hbm, dst_hbm, inv_hbm, e_hbm, wx_hbm,
              out_hbm,
              srcb_v, dstb_v, eb_v, dstw_a, dstw_b, invw_a, invw_b,
              rows_a, rows_b,
              agg_sh, isem, gsem_a, gsem_b, ssem_a, ssem_b):
    c = lax.axis_index("c")
    sid = lax.axis_index("s")
    zvec = jnp.zeros((16,), f32)
    lN = c * N
    lP = c * NPAD

    def _scale(w, invw, rows):
        for k in range(GROUPS):
            sl = pl.ds(k * 16, 16)
            eb_v[w, sl] = eb_v[w, sl] * invw[sl]
        w16 = jnp.broadcast_to(w, (16,))

        @pl.loop(0, W1, unroll=8)
        def _(r):
            sv16 = plsc.load_gather(eb_v, [w16, jnp.broadcast_to(r, (16,))])
            for q in range(D // 16):
                rsl = pl.ds(q * 16, 16)
                rows[r, rsl] = rows[r, rsl] * sv16

    @pl.loop(0, W1)
    def _(r):
        for q in range(D // 16):
            rows_a[r, pl.ds(q * 16, 16)] = zvec

    @pl.when(sid < 10)
    def _():
        @pl.loop(0, 12)
        def _(j):
            pltpu.sync_copy(rows_a, agg_sh.at[pl.ds(sid * 1000 + j * W1, W1)])

        pltpu.sync_copy(rows_a.at[pl.ds(0, 40)],
                        agg_sh.at[pl.ds(sid * 1000 + 960, 40)])

    plsc.subcore_barrier()

    def _fire(w, dstw, invw, rows, gsem):
        for k in range(GROUPS):
            sl = pl.ds(k * 16, 16)
            dstw[sl] = dstb_v[w, sl] + lP
        pltpu.async_copy(inv_hbm.at[dstw], invw, gsem)
        pltpu.async_copy(wx_hbm.at[srcb_v.at[w]], rows, gsem)

    def _wait(w, dstw, invw, rows, gsem):
        pltpu.make_async_copy(inv_hbm.at[dstw], invw, gsem).wait()
        pltpu.make_async_copy(wx_hbm.at[srcb_v.at[w]], rows, gsem).wait()

    @pl.loop(0, NBLK)
    def _(b):
        c1 = pltpu.async_copy(src_hbm.at[sid, b], srcb_v, isem)
        c2 = pltpu.async_copy(dst_hbm.at[sid, b], dstb_v, isem)
        c3 = pltpu.async_copy(e_hbm.at[c, sid, b], eb_v, isem)
        c1.wait()
        c2.wait()
        c3.wait()

        @pl.loop(0, WPB)
        def _(w):
            for k in range(GROUPS):
                sl = pl.ds(k * 16, 16)
                srcb_v[w, sl] = srcb_v[w, sl] + lN

        _fire(0, dstw_a, invw_a, rows_a, gsem_a)

        @pl.loop(0, WPB // 2)
        def _(i):
            w0 = 2 * i
            _fire(w0 + 1, dstw_b, invw_b, rows_b, gsem_b)
            _wait(w0, dstw_a, invw_a, rows_a, gsem_a)
            _scale(w0, invw_a, rows_a)
            pltpu.async_copy(rows_a, agg_sh.at[dstb_v.at[w0]], ssem_a,
                             add=True)
            _wait(w0 + 1, dstw_b, invw_b, rows_b, gsem_b)
            _scale(w0 + 1, invw_b, rows_b)
            pltpu.async_copy(rows_b, agg_sh.at[dstb_v.at[w0 + 1]], ssem_b,
                             add=True)
            pltpu.make_async_copy(rows_a, agg_sh.at[dstb_v.at[w0]],
                                  ssem_a).wait()

            @pl.when(i + 1 < WPB // 2)
            def _():
                _fire(w0 + 2, dstw_a, invw_a, rows_a, gsem_a)

            pltpu.make_async_copy(rows_b, agg_sh.at[dstb_v.at[w0 + 1]],
                                  ssem_b).wait()

    plsc.subcore_barrier()

    @pl.when(sid < 10)
    def _():
        @pl.loop(0, 5)
        def _(j):
            off = sid * 1000 + j * 200
            pltpu.sync_copy(agg_sh.at[pl.ds(off, 200)],
                            out_hbm.at[pl.ds(c * N + off, 200)])


def _sc_b(src4, dst4, inv2, e4, wx2):
    mesh = plsc.VectorSubcoreMesh(core_axis_name="c", subcore_axis_name="s")
    kern = pl.kernel(
        _scb_body,
        compiler_params=_sc_params(),
        out_type=jax.ShapeDtypeStruct((2 * N, D), f32),
        mesh=mesh,
        scratch_types=[
            pltpu.VMEM((WPB, W1), jnp.int32),
            pltpu.VMEM((WPB, W1), jnp.int32),
            pltpu.VMEM((WPB, W1), f32),
            pltpu.VMEM((W1,), jnp.int32),
            pltpu.VMEM((W1,), jnp.int32),
            pltpu.VMEM((W1,), f32),
            pltpu.VMEM((W1,), f32),
            pltpu.VMEM((W1, D), f32),
            pltpu.VMEM((W1, D), f32),
            pltpu.VMEM_SHARED((N, D), f32),
            pltpu.SemaphoreType.DMA,
            pltpu.SemaphoreType.DMA,
            pltpu.SemaphoreType.DMA,
            pltpu.SemaphoreType.DMA,
            pltpu.SemaphoreType.DMA,
        ],
    )
    return kern(src4, dst4, inv2, e4, wx2)


def kernel(sub1_text, sub1_struct, sub2_text, sub2_meta, sub3_text, sub3_meta,
           edge_index1, edge_index2, edge_index3,
           edge_type1, edge_type2, edge_type3, params):
    pairs = [
        (sub1_text, params[0], sub1_struct, params[3], edge_index1, edge_type1),
        (sub2_text, params[1], sub2_meta, params[4], edge_index2, edge_type2),
        (sub3_text, params[2], sub3_meta, params[5], edge_index3, edge_type3),
    ]
    outs = []
    for xa, pa, xb, pb, ei, et in pairs:
        wxa, tia, tja, tra = _prep(xa, pa)
        wxb, tib, tjb, trb = _prep(xb, pb)
        src4 = ei[0].reshape(NTILE, NBLK, WPB, W1)
        dst4 = ei[1].reshape(NTILE, NBLK, WPB, W1)
        et4 = et.reshape(NTILE, NBLK, WPB, W1)
        ti2 = jnp.stack([tia, tib])
        tj2 = jnp.stack([tja, tjb])
        tr2 = jnp.stack([tra, trb])
        wx2 = jnp.concatenate([wxa, wxb], axis=0)
        s2, e4 = _sc_a(src4, dst4, et4, ti2, tj2, tr2)
        inv2 = _inv(s2)
        aggflat = _sc_b(src4, dst4, inv2, e4, wx2)
        oa = _post(aggflat[:N], xa, pa[3])
        ob = _post(aggflat[N:2 * N], xb, pb[3])
        outs.append((oa, ob))
    return (outs[0][0], outs[0][1], outs[1][0], outs[1][1],
            outs[2][0], outs[2][1])
